# baseline (speedup 1.0000x reference)
import functools

import jax
import jax.numpy as jnp
from jax import lax
from jax.experimental import pallas as pl
from jax.experimental.pallas import tpu as pltpu
from jax.experimental.pallas import tpu_sc as plsc

VOCAB = 1000000
EMBED = 64
BATCH = 16384

NUM_CORES = 2
NUM_SUBCORES = 16
LANES = 16
NUM_WORKERS = NUM_CORES * NUM_SUBCORES
ROWS_PER_WORKER = BATCH // NUM_WORKERS
CHUNK = 128
NUM_CHUNKS = ROWS_PER_WORKER // CHUNK
GROUPS = ROWS_PER_WORKER // LANES


def _sc_body(x_hbm, t_hbm, table_hbm, out_hbm,
             idx_x, idx_t, rows_x, rows_t, out_v, sem):
    wid = lax.axis_index("s") * NUM_CORES + lax.axis_index("c")

    pltpu.sync_copy(x_hbm.at[wid], idx_x)
    pltpu.sync_copy(t_hbm.at[wid], idx_t)

    copies = []
    for j in range(NUM_CHUNKS):
        copies.append(pltpu.async_copy(
            table_hbm.at[idx_x.at[j]],
            rows_x.at[pl.ds(j * CHUNK, CHUNK)], sem))
        copies.append(pltpu.async_copy(
            table_hbm.at[idx_t.at[j]],
            rows_t.at[pl.ds(j * CHUNK, CHUNK)], sem))
    for c in copies:
        c.wait()

    lanes = lax.iota(jnp.int32, LANES)

    def group(g, carry):
        ridx = g * LANES + lanes
        acc = jnp.zeros((LANES,), jnp.float32)
        for d in range(EMBED):
            didx = jnp.full((LANES,), d, jnp.int32)
            gx = plsc.load_gather(rows_x, [ridx, didx])
            gt = plsc.load_gather(rows_t, [ridx, didx])
            acc = acc + gx * gt
        out_v[pl.ds(g * LANES, LANES)] = acc
        return carry

    lax.fori_loop(0, GROUPS, group, 0)

    pltpu.sync_copy(out_v, out_hbm.at[pl.ds(wid * ROWS_PER_WORKER,
                                            ROWS_PER_WORKER)])


@jax.jit
def kernel(x, t, table):
    mesh = plsc.VectorSubcoreMesh(core_axis_name="c", subcore_axis_name="s",
                                  num_cores=NUM_CORES,
                                  num_subcores=NUM_SUBCORES)
    x3 = x.reshape(NUM_WORKERS, NUM_CHUNKS, CHUNK)
    t3 = t.reshape(NUM_WORKERS, NUM_CHUNKS, CHUNK)
    run = pl.kernel(
        _sc_body,
        out_type=jax.ShapeDtypeStruct((BATCH,), jnp.float32),
        mesh=mesh,
        scratch_types=[
            pltpu.VMEM((NUM_CHUNKS, CHUNK), jnp.int32),
            pltpu.VMEM((NUM_CHUNKS, CHUNK), jnp.int32),
            pltpu.VMEM((ROWS_PER_WORKER, EMBED), jnp.float32),
            pltpu.VMEM((ROWS_PER_WORKER, EMBED), jnp.float32),
            pltpu.VMEM((ROWS_PER_WORKER,), jnp.float32),
            pltpu.SemaphoreType.DMA,
        ],
        compiler_params=pltpu.CompilerParams(needs_layout_passes=False,
                                             use_tc_tiling_on_sc=False),
    )
    return run(x3, t3, table)
